# baseline (speedup 1.0000x reference)
import jax
import jax.numpy as jnp
from jax.experimental import pallas as pl
from jax.experimental.pallas import tpu as pltpu

_CAP = 100000
_PAD_CAP = 102400
_K = 64
_Q = 1024
_QB = 64
_NCHUNK = 8
_CB = _PAD_CAP // _NCHUNK
_KNN = 50
_DELTA = 0.001
_INF16 = 0x7F80


def _dnd_kernel(q_ref, kt_ref, v_ref, out_ref, dist_ref, d16_ref):
    c = pl.program_id(1)

    q = q_ref[...]
    kt = kt_ref[0]
    qsq = jnp.sum(q * q, axis=1, keepdims=True)
    dsq = jnp.sum(kt * kt, axis=0, keepdims=True)
    prod = jax.lax.dot_general(
        q, kt, (((1,), (0,)), ((), ())),
        preferred_element_type=jnp.float32)
    dist = jnp.maximum(qsq + dsq - 2.0 * prod, 0.0)
    dist_ref[c] = dist
    bits = jax.lax.bitcast_convert_type(dist, jnp.int32)
    d16_ref[c] = jax.lax.bitcast_convert_type(
        (bits >> 16).astype(jnp.int16), jnp.bfloat16)

    @pl.when(c == _NCHUNK - 1)
    def _():
        fzero = jnp.zeros((_QB, 1), dtype=jnp.float32)

        def count16(h):
            tb = jax.lax.bitcast_convert_type(
                h.astype(jnp.int16), jnp.bfloat16)

            def cbody(k, acc):
                m = (d16_ref[k] <= tb).astype(jnp.float32)
                return acc + jnp.sum(m, axis=1, keepdims=True)
            return jax.lax.fori_loop(0, _NCHUNK, cbody, fzero)

        def body16(_, carry):
            lo, hi = carry
            mid = lo + ((hi - lo) >> 1)
            ge = count16(mid) >= float(_KNN)
            hi = jnp.where(ge, mid, hi)
            lo = jnp.where(ge, lo, mid)
            return lo, hi

        lo0 = jnp.full((_QB, 1), -1, dtype=jnp.int32)
        hi0 = jnp.full((_QB, 1), _INF16, dtype=jnp.int32)
        _, h = jax.lax.fori_loop(0, 16, body16, (lo0, hi0))

        def count_le(t):
            def cbody(k, acc):
                m = (dist_ref[k] <= t).astype(jnp.float32)
                return acc + jnp.sum(m, axis=1, keepdims=True)
            return jax.lax.fori_loop(0, _NCHUNK, cbody, fzero)

        def body32(_, carry):
            lo, hi = carry
            mid = lo + ((hi - lo) >> 1)
            t = jax.lax.bitcast_convert_type(mid, jnp.float32)
            ge = count_le(t) >= float(_KNN)
            hi = jnp.where(ge, mid, hi)
            lo = jnp.where(ge, lo, mid)
            return lo, hi

        lo1 = (h << 16) - 1
        hi1 = (h << 16) | 0xFFFF
        _, hb = jax.lax.fori_loop(0, 16, body32, (lo1, hi1))
        t50 = jax.lax.bitcast_convert_type(hb, jnp.float32)

        def wsum(k, carry):
            n_lt, n_eq, sv_eq, num_lt, den_lt = carry
            dk = dist_ref[k]
            vk = v_ref[k]
            c_lt = dk < t50
            c_eq = dk == t50
            r = jnp.where(c_lt, 1.0 / (dk + _DELTA), 0.0)
            n_lt = n_lt + jnp.sum(c_lt.astype(jnp.float32),
                                  axis=1, keepdims=True)
            n_eq = n_eq + jnp.sum(c_eq.astype(jnp.float32),
                                  axis=1, keepdims=True)
            sv_eq = sv_eq + jnp.sum(jnp.where(c_eq, vk, 0.0),
                                    axis=1, keepdims=True)
            num_lt = num_lt + jnp.sum(r * vk, axis=1, keepdims=True)
            den_lt = den_lt + jnp.sum(r, axis=1, keepdims=True)
            return n_lt, n_eq, sv_eq, num_lt, den_lt

        n_lt, n_eq, sv_eq, num_lt, den_lt = jax.lax.fori_loop(
            0, _NCHUNK, wsum, (fzero, fzero, fzero, fzero, fzero))

        w_t = 1.0 / (t50 + _DELTA)
        need = float(_KNN) - n_lt
        num = num_lt + (need / n_eq) * w_t * sv_eq
        den = den_lt + need * w_t
        out_ref[...] = num / den


def kernel(keys, dnd_keys, dnd_values):
    kt = jnp.concatenate(
        [dnd_keys.T,
         jnp.full((_K, _PAD_CAP - _CAP), 1e6, dtype=jnp.float32)], axis=1)
    kt = kt.reshape(_K, _NCHUNK, _CB).transpose(1, 0, 2)
    vals = jnp.concatenate(
        [dnd_values, jnp.zeros((_PAD_CAP - _CAP,), dtype=jnp.float32)])
    vals = vals.reshape(_NCHUNK, 1, _CB)

    return pl.pallas_call(
        _dnd_kernel,
        grid=(_Q // _QB, _NCHUNK),
        in_specs=[
            pl.BlockSpec((_QB, _K), lambda i, c: (i, 0)),
            pl.BlockSpec((1, _K, _CB), lambda i, c: (c, 0, 0)),
            pl.BlockSpec((_NCHUNK, 1, _CB), lambda i, c: (0, 0, 0)),
        ],
        out_specs=pl.BlockSpec((_QB, 1), lambda i, c: (i, 0)),
        out_shape=jax.ShapeDtypeStruct((_Q, 1), jnp.float32),
        scratch_shapes=[
            pltpu.VMEM((_NCHUNK, _QB, _CB), jnp.float32),
            pltpu.VMEM((_NCHUNK, _QB, _CB), jnp.bfloat16),
        ],
    )(keys, kt, vals)


# speedup vs baseline: 1.2101x; 1.2101x over previous
import jax
import jax.numpy as jnp
from jax.experimental import pallas as pl
from jax.experimental.pallas import tpu as pltpu

_CAP = 100000
_PAD_CAP = 102400
_K = 64
_Q = 1024
_QB = 64
_NCHUNK = 8
_CB = _PAD_CAP // _NCHUNK
_KNN = 50
_DELTA = 0.001
_INF_BITS = 0x7F800000
_BSEARCH_ITERS = 31


def _dnd_kernel(q_ref, kt_ref, v_ref, out_ref, dist_ref):
    c = pl.program_id(1)

    q = q_ref[...]
    kt = kt_ref[0]
    qsq = jnp.sum(q * q, axis=1, keepdims=True)
    dsq = jnp.sum(kt * kt, axis=0, keepdims=True)
    prod = jax.lax.dot_general(
        q, kt, (((1,), (0,)), ((), ())),
        preferred_element_type=jnp.float32)
    dist = jnp.maximum(qsq + dsq - 2.0 * prod, 0.0)
    dist_ref[c] = dist

    @pl.when(c == _NCHUNK - 1)
    def _():
        fzero = jnp.zeros((_QB, 1), dtype=jnp.float32)

        def count_le(t):
            def cbody(k, acc):
                m = (dist_ref[k] <= t).astype(jnp.float32)
                return acc + jnp.sum(m, axis=1, keepdims=True)
            return jax.lax.fori_loop(0, _NCHUNK, cbody, fzero)

        def body32(_, carry):
            lo, hi = carry
            mid = lo + ((hi - lo) >> 1)
            t = jax.lax.bitcast_convert_type(mid, jnp.float32)
            ge = count_le(t) >= float(_KNN)
            hi = jnp.where(ge, mid, hi)
            lo = jnp.where(ge, lo, mid)
            return lo, hi

        lo0 = jnp.full((_QB, 1), -1, dtype=jnp.int32)
        hi0 = jnp.full((_QB, 1), _INF_BITS, dtype=jnp.int32)
        _, hb = jax.lax.fori_loop(0, _BSEARCH_ITERS, body32, (lo0, hi0))
        t50 = jax.lax.bitcast_convert_type(hb, jnp.float32)

        def wsum(k, carry):
            n_lt, n_eq, sv_eq, num_lt, den_lt = carry
            dk = dist_ref[k]
            vk = v_ref[k]
            c_lt = dk < t50
            c_eq = dk == t50
            r = jnp.where(c_lt, 1.0 / (dk + _DELTA), 0.0)
            n_lt = n_lt + jnp.sum(c_lt.astype(jnp.float32),
                                  axis=1, keepdims=True)
            n_eq = n_eq + jnp.sum(c_eq.astype(jnp.float32),
                                  axis=1, keepdims=True)
            sv_eq = sv_eq + jnp.sum(jnp.where(c_eq, vk, 0.0),
                                    axis=1, keepdims=True)
            num_lt = num_lt + jnp.sum(r * vk, axis=1, keepdims=True)
            den_lt = den_lt + jnp.sum(r, axis=1, keepdims=True)
            return n_lt, n_eq, sv_eq, num_lt, den_lt

        n_lt, n_eq, sv_eq, num_lt, den_lt = jax.lax.fori_loop(
            0, _NCHUNK, wsum, (fzero, fzero, fzero, fzero, fzero))

        w_t = 1.0 / (t50 + _DELTA)
        need = float(_KNN) - n_lt
        num = num_lt + (need / n_eq) * w_t * sv_eq
        den = den_lt + need * w_t
        out_ref[...] = num / den


def kernel(keys, dnd_keys, dnd_values):
    kt = jnp.concatenate(
        [dnd_keys.T,
         jnp.full((_K, _PAD_CAP - _CAP), 1e6, dtype=jnp.float32)], axis=1)
    kt = kt.reshape(_K, _NCHUNK, _CB).transpose(1, 0, 2)
    vals = jnp.concatenate(
        [dnd_values, jnp.zeros((_PAD_CAP - _CAP,), dtype=jnp.float32)])
    vals = vals.reshape(_NCHUNK, 1, _CB)

    return pl.pallas_call(
        _dnd_kernel,
        grid=(_Q // _QB, _NCHUNK),
        in_specs=[
            pl.BlockSpec((_QB, _K), lambda i, c: (i, 0)),
            pl.BlockSpec((1, _K, _CB), lambda i, c: (c, 0, 0)),
            pl.BlockSpec((_NCHUNK, 1, _CB), lambda i, c: (0, 0, 0)),
        ],
        out_specs=pl.BlockSpec((_QB, 1), lambda i, c: (i, 0)),
        out_shape=jax.ShapeDtypeStruct((_Q, 1), jnp.float32),
        scratch_shapes=[
            pltpu.VMEM((_NCHUNK, _QB, _CB), jnp.float32),
        ],
    )(keys, kt, vals)
